# baseline (speedup 1.0000x reference)
import functools

import jax
import jax.numpy as jnp
from jax import lax
from jax.experimental import pallas as pl
from jax.experimental.pallas import tpu as pltpu
from jax.experimental.pallas import tpu_sc as plsc

LATENT = 32
BATCH = 16384
NC, NS, L = 2, 16, 16
NW = NC * NS
B_PER_W = BATCH // NW
CHUNK = 128
N_CHUNKS = B_PER_W // CHUNK


def _mf_body(uid_hbm, iid_hbm, ut_hbm, it_hbm, out_hbm,
             uid_v, iid_v, u_rows, v_rows, out_v, sem):
    wid = lax.axis_index("s") * NC + lax.axis_index("c")
    row_base = wid * N_CHUNKS
    base = wid * B_PER_W

    pltpu.sync_copy(uid_hbm.at[pl.ds(row_base, N_CHUNKS)], uid_v)
    pltpu.sync_copy(iid_hbm.at[pl.ds(row_base, N_CHUNKS)], iid_v)

    copies = []
    for j in range(N_CHUNKS):
        copies.append(pltpu.async_copy(
            ut_hbm.at[uid_v.at[j]], u_rows.at[pl.ds(j * CHUNK, CHUNK)], sem))
        copies.append(pltpu.async_copy(
            it_hbm.at[iid_v.at[j]], v_rows.at[pl.ds(j * CHUNK, CHUNK)], sem))
    for c in copies:
        c.wait()

    lane = lax.iota(jnp.int32, L)

    @pl.loop(0, B_PER_W // L)
    def _chunk(c):
        row = c * L + lane
        acc = jnp.zeros((L,), jnp.float32)
        for d in range(LATENT):
            dcol = jnp.full((L,), d, jnp.int32)
            u = plsc.load_gather(u_rows, [row, dcol])
            v = plsc.load_gather(v_rows, [row, dcol])
            acc = acc + u * v
        out_v[pl.ds(c * L, L)] = acc

    pltpu.sync_copy(out_v, out_hbm.at[pl.ds(base, B_PER_W)])


@jax.jit
def _mf(uid2, iid2, user_table, item_table):
    mesh = plsc.VectorSubcoreMesh(
        core_axis_name="c", subcore_axis_name="s",
        num_cores=NC, num_subcores=NS)
    run = functools.partial(
        pl.kernel,
        out_type=jax.ShapeDtypeStruct((BATCH,), jnp.float32),
        mesh=mesh,
        compiler_params=pltpu.CompilerParams(
            needs_layout_passes=False, use_tc_tiling_on_sc=False),
        scratch_types=[
            pltpu.VMEM((N_CHUNKS, CHUNK), jnp.int32),
            pltpu.VMEM((N_CHUNKS, CHUNK), jnp.int32),
            pltpu.VMEM((B_PER_W, LATENT), jnp.float32),
            pltpu.VMEM((B_PER_W, LATENT), jnp.float32),
            pltpu.VMEM((B_PER_W,), jnp.float32),
            pltpu.SemaphoreType.DMA,
        ],
    )(_mf_body)
    return run(uid2, iid2, user_table, item_table)


def kernel(user_id, item_id, user_table, item_table):
    uid2 = user_id.astype(jnp.int32).reshape(NW * N_CHUNKS, CHUNK)
    iid2 = item_id.astype(jnp.int32).reshape(NW * N_CHUNKS, CHUNK)
    return _mf(uid2, iid2, user_table, item_table)
